# Initial kernel scaffold; baseline (speedup 1.0000x reference)
import jax
import jax.numpy as jnp
from jax.experimental import pallas as pl


def kernel(gene_info, spot_position_info, spot_image, noise_key, auto_w, auto_b, dec_w, dec_b, pos0_w, pos0_b, pos1_w, pos1_b, pos2_w, pos2_b, pos3_w, pos3_b, pos4_w, pos4_b, fc0_w, fc0_b, fc1_w, fc1_b, conv1_w, conv1_b, L0b0_c1_w, L0b0_c1_b, L0b0_c2_w, L0b0_c2_b, L0b1_c1_w, L0b1_c1_b, L0b1_c2_w, L0b1_c2_b, L1b0_c1_w, L1b0_c1_b, L1b0_c2_w, L1b0_c2_b, L1b0_d_w, L1b0_d_b, L1b1_c1_w, L1b1_c1_b, L1b1_c2_w, L1b1_c2_b, L2b0_c1_w, L2b0_c1_b, L2b0_c2_w, L2b0_c2_b, L2b0_d_w, L2b0_d_b, L2b1_c1_w, L2b1_c1_b, L2b1_c2_w, L2b1_c2_b, L3b0_c1_w, L3b0_c1_b, L3b0_c2_w, L3b0_c2_b, L3b0_d_w, L3b0_d_b, L3b1_c1_w, L3b1_c1_b, L3b1_c2_w, L3b1_c2_b):
    raise NotImplementedError("write your pallas kernel here")


# speedup vs baseline: 20.3887x; 20.3887x over previous
import functools

import jax
import jax.numpy as jnp
from jax.experimental import pallas as pl
from jax.experimental.pallas import tpu as pltpu


def _wspec(shape):
    nd = len(shape)
    return pl.BlockSpec(shape, lambda i, _nd=nd: (0,) * _nd)


def _pad_hw1(x):
    bm, H, W, C = x.shape
    zr = jnp.zeros((bm, 1, W, C), x.dtype)
    x = jnp.concatenate([zr, x, zr], axis=1)
    zc = jnp.zeros((bm, H + 2, 1, C), x.dtype)
    return jnp.concatenate([zc, x, zc], axis=2)


def _conv3x3_s1(x, w_ref, b_ref, *, relu, mask=None):
    bm, H, W, C = x.shape
    N = w_ref.shape[1]
    KC = 3 * C
    xp = _pad_hw1(x)
    acc = None
    for i in range(3):
        rows = xp[:, i:i + H]
        xi = jnp.concatenate(
            [rows[:, :, 0:W], rows[:, :, 1:W + 1], rows[:, :, 2:W + 2]], axis=3)
        t = jnp.dot(xi.reshape(bm * H * W, KC), w_ref[i * KC:(i + 1) * KC, :],
                    preferred_element_type=jnp.float32)
        acc = t if acc is None else acc + t
    acc = acc + b_ref[...]
    if relu:
        acc = jnp.maximum(acc, 0.0)
    if mask is not None:
        acc = acc * mask
    return acc.astype(jnp.bfloat16).reshape(bm, H, W, N)


def _store_padded(ref, x):
    bm, Hp, Wp, C = ref.shape
    zr = jnp.zeros((bm, 1, Wp, C), ref.dtype)
    ref[:, 0:1] = zr
    ref[:, Hp - 1:Hp] = zr
    zc = jnp.zeros((bm, Hp, 1, C), ref.dtype)
    ref[:, :, 0:1] = zc
    ref[:, :, Wp - 1:Wp] = zc
    ref[:, 1:Hp - 1, 1:Wp - 1, :] = x.astype(ref.dtype)


def _conv3x3_s2(xp_ref, w_ref, b_ref, *, relu):
    bm, Hp, Wp, C = xp_ref.shape
    N = w_ref.shape[1]
    KC = 3 * C
    Ho, Wo = (Hp - 2) // 2, (Wp - 2) // 2
    acc = None
    for i in range(3):
        taps = [xp_ref[:, pl.ds(i, Ho, 2), pl.ds(j, Wo, 2), :].astype(jnp.bfloat16)
                for j in range(3)]
        xi = jnp.concatenate(taps, axis=3)
        t = jnp.dot(xi.reshape(bm * Ho * Wo, KC), w_ref[i * KC:(i + 1) * KC, :],
                    preferred_element_type=jnp.float32)
        acc = t if acc is None else acc + t
    acc = acc + b_ref[...]
    if relu:
        acc = jnp.maximum(acc, 0.0)
    return acc.astype(jnp.bfloat16).reshape(bm, Ho, Wo, N)


def _down1x1_s2(xp_ref, w_ref, b_ref):
    bm, Hp, Wp, C = xp_ref.shape
    N = w_ref.shape[1]
    Ho, Wo = (Hp - 2) // 2, (Wp - 2) // 2
    xs = xp_ref[:, pl.ds(1, Ho, 2), pl.ds(1, Wo, 2), :].astype(jnp.bfloat16)
    acc = jnp.dot(xs.reshape(bm * Ho * Wo, C), w_ref[...],
                  preferred_element_type=jnp.float32) + b_ref[...]
    return acc.astype(jnp.bfloat16).reshape(bm, Ho, Wo, N)


def _conv3x3_s2_val(x, w_ref, b_ref, *, relu):
    bm, H, W, C = x.shape
    N = w_ref.shape[1]
    KC = 3 * C
    Ho, Wo = H // 2, W // 2
    xp = _pad_hw1(x)
    xr = xp.reshape(bm, (H + 2) // 2, 2, W + 2, C)
    acc = None
    for i in range(3):
        rows = xr[:, (i // 2):(i // 2) + Ho, i % 2]
        wcols = []
        for j in range(3):
            pieces = [rows[:, :, j + 2 * x0:j + 2 * x0 + 1, :] for x0 in range(Wo)]
            wcols.append(jnp.concatenate(pieces, axis=2) if Wo > 1 else pieces[0])
        xi = jnp.concatenate(wcols, axis=3)
        t = jnp.dot(xi.reshape(bm * Ho * Wo, KC), w_ref[i * KC:(i + 1) * KC, :],
                    preferred_element_type=jnp.float32)
        acc = t if acc is None else acc + t
    acc = acc + b_ref[...]
    if relu:
        acc = jnp.maximum(acc, 0.0)
    return acc.astype(jnp.bfloat16).reshape(bm, Ho, Wo, N)


def _down1x1_s2_val(x, w_ref, b_ref):
    bm, H, W, C = x.shape
    N = w_ref.shape[1]
    Ho, Wo = H // 2, W // 2
    rows = x.reshape(bm, Ho, 2, W, C)[:, :, 0]
    pieces = [rows[:, :, 2 * x0:2 * x0 + 1, :] for x0 in range(Wo)]
    xs = jnp.concatenate(pieces, axis=2) if Wo > 1 else pieces[0]
    acc = jnp.dot(xs.reshape(bm * Ho * Wo, C), w_ref[...],
                  preferred_element_type=jnp.float32) + b_ref[...]
    return acc.astype(jnp.bfloat16).reshape(bm, Ho, Wo, N)


def _block_s2_val(x, w1, b1, w2, b2, wd, bd):
    o = _conv3x3_s2_val(x, w1, b1, relu=True)
    o = _conv3x3_s1(o, w2, b2, relu=False)
    idn = _down1x1_s2_val(x, wd, bd)
    return jnp.maximum(o + idn, 0.0)


def _block_s1(x, w1, b1, w2, b2, mask=None):
    o = _conv3x3_s1(x, w1, b1, relu=True, mask=mask)
    o = _conv3x3_s1(o, w2, b2, relu=False, mask=mask)
    return jnp.maximum(o + x, 0.0)


def _block_s2(x, sref, w1, b1, w2, b2, wd, bd):
    _store_padded(sref, x)
    o = _conv3x3_s2(sref, w1, b1, relu=True)
    o = _conv3x3_s1(o, w2, b2, relu=False)
    idn = _down1x1_s2(sref, wd, bd)
    return jnp.maximum(o + idn, 0.0)


def _maxpool3x3_s2(xp_ref):
    bm, Hp, Wp, C = xp_ref.shape
    Ho, Wo = (Hp - 2) // 2, (Wp - 2) // 2
    out = None
    for i in range(3):
        for j in range(3):
            t = xp_ref[:, pl.ds(i, Ho, 2), pl.ds(j, Wo, 2), :]
            out = t if out is None else jnp.maximum(out, t)
    return out.astype(jnp.bfloat16)


def _front_kernel(cols_ref, c1w, c1b,
                  a0c1w, a0c1b, a0c2w, a0c2b, a1c1w, a1c1b, a1c2w, a1c2b,
                  b0c1w, b0c1b, b0c2w, b0c2b, b0dw, b0db,
                  b1c1w, b1c1b, b1c2w, b1c2b, o_ref, sp_ref, s1_ref, *, bm):
    mask = (jax.lax.broadcasted_iota(jnp.int32, (1, 128), 1) < 64).astype(jnp.float32)
    acc = jnp.dot(cols_ref[...], c1w[...], preferred_element_type=jnp.float32)
    acc = jnp.maximum(acc + c1b[...], 0.0) * mask
    x = acc.astype(jnp.bfloat16).reshape(bm, 32, 32, 128)
    _store_padded(sp_ref, x)
    x = _maxpool3x3_s2(sp_ref)
    x = _block_s1(x, a0c1w, a0c1b, a0c2w, a0c2b, mask=mask)
    x = _block_s1(x, a1c1w, a1c1b, a1c2w, a1c2b, mask=mask)
    x = _block_s2(x, s1_ref, b0c1w, b0c1b, b0c2w, b0c2b, b0dw, b0db)
    x = _block_s1(x, b1c1w, b1c1b, b1c2w, b1c2b)
    o_ref[...] = x


def _back_kernel(x_ref, auto_ref, pos_ref,
                 c0c1w, c0c1b, c0c2w, c0c2b, c0dw, c0db,
                 c1c1w, c1c1b, c1c2w, c1c2b,
                 d0c1w, d0c1b, d0c2w, d0c2b, d0dw, d0db,
                 d1c1w, d1c1b, d1c2w, d1c2b,
                 fc0w, fc0b, fc1w, fc1b,
                 p0w, p0b, p1w, p1b, p2w, p2b, p3w, p3b, p4w, p4b,
                 aw, ab, dw, db, o_ref, s2_ref, *, bm):
    x = x_ref[...]
    x = _block_s2(x, s2_ref, c0c1w, c0c1b, c0c2w, c0c2b, c0dw, c0db)
    x = _block_s1(x, c1c1w, c1c1b, c1c2w, c1c2b)
    x = _block_s2_val(x, d0c1w, d0c1b, d0c2w, d0c2b, d0dw, d0db)
    x = _block_s1(x, d1c1w, d1c1b, d1c2w, d1c2b)
    g = jnp.mean(x.astype(jnp.float32), axis=(1, 2))
    h = g.astype(jnp.bfloat16)
    a = jnp.maximum(jnp.dot(h, fc0w[...], preferred_element_type=jnp.float32)
                    + fc0b[...], 0.0)
    img = jnp.maximum(jnp.dot(a.astype(jnp.bfloat16), fc1w[...],
                              preferred_element_type=jnp.float32) + fc1b[...], 0.0)
    h = pos_ref[...].astype(jnp.bfloat16)
    for w_r, b_r in ((p0w, p0b), (p1w, p1b), (p2w, p2b), (p3w, p3b), (p4w, p4b)):
        acc = jnp.maximum(jnp.dot(h, w_r[...], preferred_element_type=jnp.float32)
                          + b_r[...], 0.0)
        h = acc.astype(jnp.bfloat16)
    pos = acc
    a = jnp.maximum(jnp.dot(auto_ref[...].astype(jnp.bfloat16), aw[...],
                            preferred_element_type=jnp.float32) + ab[...], 0.0)
    enc = a + pos + img
    d = jnp.dot(enc.astype(jnp.bfloat16), dw[...], preferred_element_type=jnp.float32)
    o_ref[...] = jnp.maximum(d + db[...], 0.0)


def kernel(gene_info, spot_position_info, spot_image, noise_key, auto_w, auto_b,
           dec_w, dec_b, pos0_w, pos0_b, pos1_w, pos1_b, pos2_w, pos2_b,
           pos3_w, pos3_b, pos4_w, pos4_b, fc0_w, fc0_b, fc1_w, fc1_b,
           conv1_w, conv1_b,
           L0b0_c1_w, L0b0_c1_b, L0b0_c2_w, L0b0_c2_b,
           L0b1_c1_w, L0b1_c1_b, L0b1_c2_w, L0b1_c2_b,
           L1b0_c1_w, L1b0_c1_b, L1b0_c2_w, L1b0_c2_b, L1b0_d_w, L1b0_d_b,
           L1b1_c1_w, L1b1_c1_b, L1b1_c2_w, L1b1_c2_b,
           L2b0_c1_w, L2b0_c1_b, L2b0_c2_w, L2b0_c2_b, L2b0_d_w, L2b0_d_b,
           L2b1_c1_w, L2b1_c1_b, L2b1_c2_w, L2b1_c2_b,
           L3b0_c1_w, L3b0_c1_b, L3b0_c2_w, L3b0_c2_b, L3b0_d_w, L3b0_d_b,
           L3b1_c1_w, L3b1_c1_b, L3b1_c2_w, L3b1_c2_b):
    B = gene_info.shape[0]

    gmean = jnp.mean(gene_info)
    gstd = jnp.std(gene_info, ddof=1)
    noise = jax.random.normal(noise_key, gene_info.shape, jnp.float32) * gstd + gmean
    auto_in = gene_info + jnp.maximum(noise * 0.1, 0.0)

    x = jnp.transpose(spot_image, (0, 2, 3, 1)).astype(jnp.bfloat16)
    xp = jnp.pad(x, ((0, 0), (3, 3), (3, 3), (0, 0)))
    patches = [xp[:, i:i + 63:2, j:j + 63:2, :] for i in range(7) for j in range(7)]
    cols = jnp.concatenate(patches, axis=-1).reshape(B * 1024, 147)

    def cinpad(w):
        return jnp.pad(w.reshape(3, 3, 64, -1),
                       ((0, 0), (0, 0), (0, 64), (0, 0))).reshape(1152, -1)

    def rb(b):
        return b.reshape(1, -1).astype(jnp.float32)

    bm1 = 16 if B % 16 == 0 else B
    f_in = [cols, conv1_w, rb(conv1_b),
            cinpad(L0b0_c1_w), rb(L0b0_c1_b), cinpad(L0b0_c2_w), rb(L0b0_c2_b),
            cinpad(L0b1_c1_w), rb(L0b1_c1_b), cinpad(L0b1_c2_w), rb(L0b1_c2_b),
            cinpad(L1b0_c1_w), rb(L1b0_c1_b), L1b0_c2_w, rb(L1b0_c2_b),
            jnp.pad(L1b0_d_w, ((0, 64), (0, 0))), rb(L1b0_d_b),
            L1b1_c1_w, rb(L1b1_c1_b), L1b1_c2_w, rb(L1b1_c2_b)]
    front = pl.pallas_call(
        functools.partial(_front_kernel, bm=bm1),
        out_shape=jax.ShapeDtypeStruct((B, 8, 8, 128), jnp.bfloat16),
        grid=(B // bm1,),
        in_specs=[pl.BlockSpec((bm1 * 1024, 147), lambda i: (i, 0))]
                 + [_wspec(a.shape) for a in f_in[1:]],
        out_specs=pl.BlockSpec((bm1, 8, 8, 128), lambda i: (i, 0, 0, 0)),
        scratch_shapes=[pltpu.VMEM((bm1, 34, 34, 128), jnp.float32),
                        pltpu.VMEM((bm1, 18, 18, 128), jnp.float32)],
        compiler_params=pltpu.CompilerParams(dimension_semantics=("parallel",)),
    )(*f_in)

    bm2 = 128 if B % 128 == 0 else B
    b_in = [front, auto_in, spot_position_info,
            L2b0_c1_w, rb(L2b0_c1_b), L2b0_c2_w, rb(L2b0_c2_b),
            L2b0_d_w, rb(L2b0_d_b),
            L2b1_c1_w, rb(L2b1_c1_b), L2b1_c2_w, rb(L2b1_c2_b),
            L3b0_c1_w, rb(L3b0_c1_b), L3b0_c2_w, rb(L3b0_c2_b),
            L3b0_d_w, rb(L3b0_d_b),
            L3b1_c1_w, rb(L3b1_c1_b), L3b1_c2_w, rb(L3b1_c2_b),
            fc0_w, rb(fc0_b), fc1_w, rb(fc1_b),
            pos0_w, rb(pos0_b), pos1_w, rb(pos1_b), pos2_w, rb(pos2_b),
            pos3_w, rb(pos3_b), pos4_w, rb(pos4_b),
            auto_w, rb(auto_b), dec_w, rb(dec_b)]
    out = pl.pallas_call(
        functools.partial(_back_kernel, bm=bm2),
        out_shape=jax.ShapeDtypeStruct((B, 256), jnp.float32),
        grid=(B // bm2,),
        in_specs=[pl.BlockSpec((bm2, 8, 8, 128), lambda i: (i, 0, 0, 0)),
                  pl.BlockSpec((bm2, 256), lambda i: (i, 0)),
                  pl.BlockSpec((bm2, 8), lambda i: (i, 0))]
                 + [_wspec(a.shape) for a in b_in[3:]],
        out_specs=pl.BlockSpec((bm2, 256), lambda i: (i, 0)),
        scratch_shapes=[pltpu.VMEM((bm2, 10, 10, 128), jnp.float32)],
        compiler_params=pltpu.CompilerParams(dimension_semantics=("parallel",)),
    )(*b_in)
    return out


# speedup vs baseline: 72.2300x; 3.5427x over previous
import functools

import jax
import jax.numpy as jnp
from jax.experimental import pallas as pl
from jax.experimental.pallas import tpu as pltpu


def _wspec(shape):
    nd = len(shape)
    return pl.BlockSpec(shape, lambda i, _nd=nd: (0,) * _nd)


def _pad_hw1(x):
    bm, H, W, C = x.shape
    zr = jnp.zeros((bm, 1, W, C), x.dtype)
    x = jnp.concatenate([zr, x, zr], axis=1)
    zc = jnp.zeros((bm, H + 2, 1, C), x.dtype)
    return jnp.concatenate([zc, x, zc], axis=2)


def _conv3x3_s1(x, w_ref, b_ref, *, relu, mask=None):
    bm, H, W, C = x.shape
    N = w_ref.shape[1]
    KC = 3 * C
    xp = _pad_hw1(x)
    acc = None
    for i in range(3):
        rows = xp[:, i:i + H]
        xi = jnp.concatenate(
            [rows[:, :, 0:W], rows[:, :, 1:W + 1], rows[:, :, 2:W + 2]], axis=3)
        t = jnp.dot(xi.reshape(bm * H * W, KC), w_ref[i * KC:(i + 1) * KC, :],
                    preferred_element_type=jnp.float32)
        acc = t if acc is None else acc + t
    acc = acc + b_ref[...]
    if relu:
        acc = jnp.maximum(acc, 0.0)
    if mask is not None:
        acc = acc * mask
    return acc.astype(jnp.bfloat16).reshape(bm, H, W, N)


def _store_padded(ref, x):
    bm, Hp, Wp, C = ref.shape
    zr = jnp.zeros((bm, 1, Wp, C), ref.dtype)
    ref[:, 0:1] = zr
    ref[:, Hp - 1:Hp] = zr
    zc = jnp.zeros((bm, Hp, 1, C), ref.dtype)
    ref[:, :, 0:1] = zc
    ref[:, :, Wp - 1:Wp] = zc
    ref[:, 1:Hp - 1, 1:Wp - 1, :] = x.astype(ref.dtype)


def _conv3x3_s2(xp_ref, w_ref, b_ref, *, relu):
    bm, Hp, Wp, C = xp_ref.shape
    N = w_ref.shape[1]
    KC = 3 * C
    Ho, Wo = (Hp - 2) // 2, (Wp - 2) // 2
    acc = None
    for i in range(3):
        taps = [xp_ref[:, pl.ds(i, Ho, 2), pl.ds(j, Wo, 2), :].astype(jnp.bfloat16)
                for j in range(3)]
        xi = jnp.concatenate(taps, axis=3)
        t = jnp.dot(xi.reshape(bm * Ho * Wo, KC), w_ref[i * KC:(i + 1) * KC, :],
                    preferred_element_type=jnp.float32)
        acc = t if acc is None else acc + t
    acc = acc + b_ref[...]
    if relu:
        acc = jnp.maximum(acc, 0.0)
    return acc.astype(jnp.bfloat16).reshape(bm, Ho, Wo, N)


def _down1x1_s2(xp_ref, w_ref, b_ref):
    bm, Hp, Wp, C = xp_ref.shape
    N = w_ref.shape[1]
    Ho, Wo = (Hp - 2) // 2, (Wp - 2) // 2
    xs = xp_ref[:, pl.ds(1, Ho, 2), pl.ds(1, Wo, 2), :].astype(jnp.bfloat16)
    acc = jnp.dot(xs.reshape(bm * Ho * Wo, C), w_ref[...],
                  preferred_element_type=jnp.float32) + b_ref[...]
    return acc.astype(jnp.bfloat16).reshape(bm, Ho, Wo, N)


def _conv3x3_s2_val(x, w_ref, b_ref, *, relu):
    bm, H, W, C = x.shape
    N = w_ref.shape[1]
    KC = 3 * C
    Ho, Wo = H // 2, W // 2
    xp = _pad_hw1(x)
    xr = xp.reshape(bm, (H + 2) // 2, 2, W + 2, C)
    acc = None
    for i in range(3):
        rows = xr[:, (i // 2):(i // 2) + Ho, i % 2]
        wcols = []
        for j in range(3):
            pieces = [rows[:, :, j + 2 * x0:j + 2 * x0 + 1, :] for x0 in range(Wo)]
            wcols.append(jnp.concatenate(pieces, axis=2) if Wo > 1 else pieces[0])
        xi = jnp.concatenate(wcols, axis=3)
        t = jnp.dot(xi.reshape(bm * Ho * Wo, KC), w_ref[i * KC:(i + 1) * KC, :],
                    preferred_element_type=jnp.float32)
        acc = t if acc is None else acc + t
    acc = acc + b_ref[...]
    if relu:
        acc = jnp.maximum(acc, 0.0)
    return acc.astype(jnp.bfloat16).reshape(bm, Ho, Wo, N)


def _down1x1_s2_val(x, w_ref, b_ref):
    bm, H, W, C = x.shape
    N = w_ref.shape[1]
    Ho, Wo = H // 2, W // 2
    rows = x.reshape(bm, Ho, 2, W, C)[:, :, 0]
    pieces = [rows[:, :, 2 * x0:2 * x0 + 1, :] for x0 in range(Wo)]
    xs = jnp.concatenate(pieces, axis=2) if Wo > 1 else pieces[0]
    acc = jnp.dot(xs.reshape(bm * Ho * Wo, C), w_ref[...],
                  preferred_element_type=jnp.float32) + b_ref[...]
    return acc.astype(jnp.bfloat16).reshape(bm, Ho, Wo, N)


def _block_s2_val(x, w1, b1, w2, b2, wd, bd):
    o = _conv3x3_s2_val(x, w1, b1, relu=True)
    o = _conv3x3_s1(o, w2, b2, relu=False)
    idn = _down1x1_s2_val(x, wd, bd)
    return jnp.maximum(o + idn, 0.0)


def _block_s1(x, w1, b1, w2, b2, mask=None):
    o = _conv3x3_s1(x, w1, b1, relu=True, mask=mask)
    o = _conv3x3_s1(o, w2, b2, relu=False, mask=mask)
    return jnp.maximum(o + x, 0.0)


def _block_s2(x, sref, w1, b1, w2, b2, wd, bd):
    _store_padded(sref, x)
    o = _conv3x3_s2(sref, w1, b1, relu=True)
    o = _conv3x3_s1(o, w2, b2, relu=False)
    idn = _down1x1_s2(sref, wd, bd)
    return jnp.maximum(o + idn, 0.0)


def _maxpool3x3_s2(xp_ref):
    bm, Hp, Wp, C = xp_ref.shape
    Ho, Wo = (Hp - 2) // 2, (Wp - 2) // 2
    out = None
    for i in range(3):
        for j in range(3):
            t = xp_ref[:, pl.ds(i, Ho, 2), pl.ds(j, Wo, 2), :]
            out = t if out is None else jnp.maximum(out, t)
    return out.astype(jnp.bfloat16)


def _front_kernel(img_ref, c1w, c1b, c1m,
                  a0c1w, a0c1b, a0c2w, a0c2b, a1c1w, a1c1b, a1c2w, a1c2b,
                  b0c1w, b0c1b, b0c2w, b0c2b, b0dw, b0db,
                  b1c1w, b1c1b, b1c2w, b1c2b, o_ref, sp_ref, s1_ref, *, bm):
    mask = (jax.lax.broadcasted_iota(jnp.int32, (1, 128), 1) < 64).astype(jnp.float32)
    x = img_ref[...]
    pieces = [x[:, dY:dY + 32, dG:dG + 8, :] for dY in range(4) for dG in range(2)]
    xi = jnp.concatenate(pieces, axis=-1).reshape(bm * 256, 384)
    acc = jnp.dot(xi, c1w[...], preferred_element_type=jnp.float32)
    acc = jnp.maximum(acc + c1b[...], 0.0) * c1m[...]
    x = acc.astype(jnp.bfloat16).reshape(bm, 32, 32, 128)
    _store_padded(sp_ref, x)
    x = _maxpool3x3_s2(sp_ref)
    x = _block_s1(x, a0c1w, a0c1b, a0c2w, a0c2b, mask=mask)
    x = _block_s1(x, a1c1w, a1c1b, a1c2w, a1c2b, mask=mask)
    x = _block_s2(x, s1_ref, b0c1w, b0c1b, b0c2w, b0c2b, b0dw, b0db)
    x = _block_s1(x, b1c1w, b1c1b, b1c2w, b1c2b)
    o_ref[...] = x


def _back_kernel(x_ref, auto_ref, pos_ref,
                 c0c1w, c0c1b, c0c2w, c0c2b, c0dw, c0db,
                 c1c1w, c1c1b, c1c2w, c1c2b,
                 d0c1w, d0c1b, d0c2w, d0c2b, d0dw, d0db,
                 d1c1w, d1c1b, d1c2w, d1c2b,
                 fc0w, fc0b, fc1w, fc1b,
                 p0w, p0b, p1w, p1b, p2w, p2b, p3w, p3b, p4w, p4b,
                 aw, ab, dw, db, o_ref, s2_ref, *, bm):
    x = x_ref[...]
    x = _block_s2(x, s2_ref, c0c1w, c0c1b, c0c2w, c0c2b, c0dw, c0db)
    x = _block_s1(x, c1c1w, c1c1b, c1c2w, c1c2b)
    x = _block_s2_val(x, d0c1w, d0c1b, d0c2w, d0c2b, d0dw, d0db)
    x = _block_s1(x, d1c1w, d1c1b, d1c2w, d1c2b)
    g = jnp.mean(x.astype(jnp.float32), axis=(1, 2))
    h = g.astype(jnp.bfloat16)
    a = jnp.maximum(jnp.dot(h, fc0w[...], preferred_element_type=jnp.float32)
                    + fc0b[...], 0.0)
    img = jnp.maximum(jnp.dot(a.astype(jnp.bfloat16), fc1w[...],
                              preferred_element_type=jnp.float32) + fc1b[...], 0.0)
    h = pos_ref[...].astype(jnp.bfloat16)
    for w_r, b_r in ((p0w, p0b), (p1w, p1b), (p2w, p2b), (p3w, p3b), (p4w, p4b)):
        acc = jnp.maximum(jnp.dot(h, w_r[...], preferred_element_type=jnp.float32)
                          + b_r[...], 0.0)
        h = acc.astype(jnp.bfloat16)
    pos = acc
    a = jnp.maximum(jnp.dot(auto_ref[...].astype(jnp.bfloat16), aw[...],
                            preferred_element_type=jnp.float32) + ab[...], 0.0)
    enc = a + pos + img
    d = jnp.dot(enc.astype(jnp.bfloat16), dw[...], preferred_element_type=jnp.float32)
    o_ref[...] = jnp.maximum(d + db[...], 0.0)


def kernel(gene_info, spot_position_info, spot_image, noise_key, auto_w, auto_b,
           dec_w, dec_b, pos0_w, pos0_b, pos1_w, pos1_b, pos2_w, pos2_b,
           pos3_w, pos3_b, pos4_w, pos4_b, fc0_w, fc0_b, fc1_w, fc1_b,
           conv1_w, conv1_b,
           L0b0_c1_w, L0b0_c1_b, L0b0_c2_w, L0b0_c2_b,
           L0b1_c1_w, L0b1_c1_b, L0b1_c2_w, L0b1_c2_b,
           L1b0_c1_w, L1b0_c1_b, L1b0_c2_w, L1b0_c2_b, L1b0_d_w, L1b0_d_b,
           L1b1_c1_w, L1b1_c1_b, L1b1_c2_w, L1b1_c2_b,
           L2b0_c1_w, L2b0_c1_b, L2b0_c2_w, L2b0_c2_b, L2b0_d_w, L2b0_d_b,
           L2b1_c1_w, L2b1_c1_b, L2b1_c2_w, L2b1_c2_b,
           L3b0_c1_w, L3b0_c1_b, L3b0_c2_w, L3b0_c2_b, L3b0_d_w, L3b0_d_b,
           L3b1_c1_w, L3b1_c1_b, L3b1_c2_w, L3b1_c2_b):
    B = gene_info.shape[0]

    gmean = jnp.mean(gene_info)
    gstd = jnp.std(gene_info, ddof=1)
    noise = jax.random.normal(noise_key, gene_info.shape, jnp.float32) * gstd + gmean
    auto_in = gene_info + jnp.maximum(noise * 0.1, 0.0)

    x = jnp.transpose(spot_image, (0, 2, 3, 1)).astype(jnp.bfloat16)
    xp = jnp.pad(x, ((0, 0), (3, 3), (3, 5), (0, 0)))
    xr = xp.reshape(B, 35, 2, 9, 8, 3).transpose(0, 1, 3, 2, 4, 5)
    xr = xr.reshape(B, 35, 9, 48)

    w7 = conv1_w.reshape(7, 7, 3, 128)
    wjp = jnp.pad(w7, ((0, 1), (6, 9), (0, 0), (0, 0)))
    dYv = jnp.arange(4)[:, None, None, None]
    dGv = jnp.arange(2)[None, :, None, None]
    phrv = jnp.arange(2)[None, None, :, None]
    phcv = jnp.arange(8)[None, None, None, :]
    i_idx = jnp.broadcast_to(2 * dYv + phrv, (4, 2, 2, 8))
    w_u = [wjp[i_idx, jnp.broadcast_to(8 * dGv + phcv - 2 * u + 6, (4, 2, 2, 8))]
           .reshape(384, 128) for u in range(4)]
    c1w384 = jnp.stack(w_u, axis=1).reshape(384, 512)
    c1b512 = jnp.tile(conv1_b.reshape(1, 128).astype(jnp.float32), (1, 4))
    c1m512 = ((jnp.arange(512) % 128) < 64).astype(jnp.float32).reshape(1, 512)

    def cinpad(w):
        return jnp.pad(w.reshape(3, 3, 64, -1),
                       ((0, 0), (0, 0), (0, 64), (0, 0))).reshape(1152, -1)

    def rb(b):
        return b.reshape(1, -1).astype(jnp.float32)

    bm1 = 16 if B % 16 == 0 else B
    f_in = [xr, c1w384, c1b512, c1m512,
            cinpad(L0b0_c1_w), rb(L0b0_c1_b), cinpad(L0b0_c2_w), rb(L0b0_c2_b),
            cinpad(L0b1_c1_w), rb(L0b1_c1_b), cinpad(L0b1_c2_w), rb(L0b1_c2_b),
            cinpad(L1b0_c1_w), rb(L1b0_c1_b), L1b0_c2_w, rb(L1b0_c2_b),
            jnp.pad(L1b0_d_w, ((0, 64), (0, 0))), rb(L1b0_d_b),
            L1b1_c1_w, rb(L1b1_c1_b), L1b1_c2_w, rb(L1b1_c2_b)]
    front = pl.pallas_call(
        functools.partial(_front_kernel, bm=bm1),
        out_shape=jax.ShapeDtypeStruct((B, 8, 8, 128), jnp.bfloat16),
        grid=(B // bm1,),
        in_specs=[pl.BlockSpec((bm1, 35, 9, 48), lambda i: (i, 0, 0, 0))]
                 + [_wspec(a.shape) for a in f_in[1:]],
        out_specs=pl.BlockSpec((bm1, 8, 8, 128), lambda i: (i, 0, 0, 0)),
        scratch_shapes=[pltpu.VMEM((bm1, 34, 34, 128), jnp.float32),
                        pltpu.VMEM((bm1, 18, 18, 128), jnp.float32)],
        compiler_params=pltpu.CompilerParams(dimension_semantics=("parallel",)),
    )(*f_in)

    bm2 = 128 if B % 128 == 0 else B
    b_in = [front, auto_in, spot_position_info,
            L2b0_c1_w, rb(L2b0_c1_b), L2b0_c2_w, rb(L2b0_c2_b),
            L2b0_d_w, rb(L2b0_d_b),
            L2b1_c1_w, rb(L2b1_c1_b), L2b1_c2_w, rb(L2b1_c2_b),
            L3b0_c1_w, rb(L3b0_c1_b), L3b0_c2_w, rb(L3b0_c2_b),
            L3b0_d_w, rb(L3b0_d_b),
            L3b1_c1_w, rb(L3b1_c1_b), L3b1_c2_w, rb(L3b1_c2_b),
            fc0_w, rb(fc0_b), fc1_w, rb(fc1_b),
            pos0_w, rb(pos0_b), pos1_w, rb(pos1_b), pos2_w, rb(pos2_b),
            pos3_w, rb(pos3_b), pos4_w, rb(pos4_b),
            auto_w, rb(auto_b), dec_w, rb(dec_b)]
    out = pl.pallas_call(
        functools.partial(_back_kernel, bm=bm2),
        out_shape=jax.ShapeDtypeStruct((B, 256), jnp.float32),
        grid=(B // bm2,),
        in_specs=[pl.BlockSpec((bm2, 8, 8, 128), lambda i: (i, 0, 0, 0)),
                  pl.BlockSpec((bm2, 256), lambda i: (i, 0)),
                  pl.BlockSpec((bm2, 8), lambda i: (i, 0))]
                 + [_wspec(a.shape) for a in b_in[3:]],
        out_specs=pl.BlockSpec((bm2, 256), lambda i: (i, 0)),
        scratch_shapes=[pltpu.VMEM((bm2, 10, 10, 128), jnp.float32)],
        compiler_params=pltpu.CompilerParams(dimension_semantics=("parallel",)),
    )(*b_in)
    return out


# speedup vs baseline: 84.2301x; 1.1661x over previous
import functools

import jax
import jax.numpy as jnp
from jax.experimental import pallas as pl
from jax.experimental.pallas import tpu as pltpu


def _wspec(shape):
    nd = len(shape)
    return pl.BlockSpec(shape, lambda i, _nd=nd: (0,) * _nd)


def _pad_hw1(x):
    bm, H, W, C = x.shape
    zr = jnp.zeros((bm, 1, W, C), x.dtype)
    x = jnp.concatenate([zr, x, zr], axis=1)
    zc = jnp.zeros((bm, H + 2, 1, C), x.dtype)
    return jnp.concatenate([zc, x, zc], axis=2)


def _conv3x3_s1(x, w_ref, b_ref, *, relu, mask=None):
    bm, H, W, C = x.shape
    N = w_ref.shape[1]
    KC = 3 * C
    xp = _pad_hw1(x)
    acc = None
    for i in range(3):
        rows = xp[:, i:i + H]
        xi = jnp.concatenate(
            [rows[:, :, 0:W], rows[:, :, 1:W + 1], rows[:, :, 2:W + 2]], axis=3)
        t = jnp.dot(xi.reshape(bm * H * W, KC), w_ref[i * KC:(i + 1) * KC, :],
                    preferred_element_type=jnp.float32)
        acc = t if acc is None else acc + t
    acc = acc + b_ref[...]
    if relu:
        acc = jnp.maximum(acc, 0.0)
    if mask is not None:
        acc = acc * mask
    return acc.astype(jnp.bfloat16).reshape(bm, H, W, N)


def _store_padded(ref, x):
    bm, Hp, Wp, C = ref.shape
    zr = jnp.zeros((bm, 1, Wp, C), ref.dtype)
    ref[:, 0:1] = zr
    ref[:, Hp - 1:Hp] = zr
    zc = jnp.zeros((bm, Hp, 1, C), ref.dtype)
    ref[:, :, 0:1] = zc
    ref[:, :, Wp - 1:Wp] = zc
    ref[:, 1:Hp - 1, 1:Wp - 1, :] = x.astype(ref.dtype)


def _conv3x3_s2(xp_ref, w_ref, b_ref, *, relu):
    bm, Hp, Wp, C = xp_ref.shape
    N = w_ref.shape[1]
    KC = 3 * C
    Ho, Wo = (Hp - 2) // 2, (Wp - 2) // 2
    acc = None
    for i in range(3):
        taps = [xp_ref[:, pl.ds(i, Ho, 2), pl.ds(j, Wo, 2), :].astype(jnp.bfloat16)
                for j in range(3)]
        xi = jnp.concatenate(taps, axis=3)
        t = jnp.dot(xi.reshape(bm * Ho * Wo, KC), w_ref[i * KC:(i + 1) * KC, :],
                    preferred_element_type=jnp.float32)
        acc = t if acc is None else acc + t
    acc = acc + b_ref[...]
    if relu:
        acc = jnp.maximum(acc, 0.0)
    return acc.astype(jnp.bfloat16).reshape(bm, Ho, Wo, N)


def _down1x1_s2(xp_ref, w_ref, b_ref):
    bm, Hp, Wp, C = xp_ref.shape
    N = w_ref.shape[1]
    Ho, Wo = (Hp - 2) // 2, (Wp - 2) // 2
    xs = xp_ref[:, pl.ds(1, Ho, 2), pl.ds(1, Wo, 2), :].astype(jnp.bfloat16)
    acc = jnp.dot(xs.reshape(bm * Ho * Wo, C), w_ref[...],
                  preferred_element_type=jnp.float32) + b_ref[...]
    return acc.astype(jnp.bfloat16).reshape(bm, Ho, Wo, N)


def _conv3x3_s2_val(x, w_ref, b_ref, *, relu):
    bm, H, W, C = x.shape
    N = w_ref.shape[1]
    KC = 3 * C
    Ho, Wo = H // 2, W // 2
    xp = _pad_hw1(x)
    xr = xp.reshape(bm, (H + 2) // 2, 2, W + 2, C)
    acc = None
    for i in range(3):
        rows = xr[:, (i // 2):(i // 2) + Ho, i % 2]
        wcols = []
        for j in range(3):
            pieces = [rows[:, :, j + 2 * x0:j + 2 * x0 + 1, :] for x0 in range(Wo)]
            wcols.append(jnp.concatenate(pieces, axis=2) if Wo > 1 else pieces[0])
        xi = jnp.concatenate(wcols, axis=3)
        t = jnp.dot(xi.reshape(bm * Ho * Wo, KC), w_ref[i * KC:(i + 1) * KC, :],
                    preferred_element_type=jnp.float32)
        acc = t if acc is None else acc + t
    acc = acc + b_ref[...]
    if relu:
        acc = jnp.maximum(acc, 0.0)
    return acc.astype(jnp.bfloat16).reshape(bm, Ho, Wo, N)


def _down1x1_s2_val(x, w_ref, b_ref):
    bm, H, W, C = x.shape
    N = w_ref.shape[1]
    Ho, Wo = H // 2, W // 2
    rows = x.reshape(bm, Ho, 2, W, C)[:, :, 0]
    pieces = [rows[:, :, 2 * x0:2 * x0 + 1, :] for x0 in range(Wo)]
    xs = jnp.concatenate(pieces, axis=2) if Wo > 1 else pieces[0]
    acc = jnp.dot(xs.reshape(bm * Ho * Wo, C), w_ref[...],
                  preferred_element_type=jnp.float32) + b_ref[...]
    return acc.astype(jnp.bfloat16).reshape(bm, Ho, Wo, N)


def _block_s2_val(x, w1, b1, w2, b2, wd, bd):
    o = _conv3x3_s2_val(x, w1, b1, relu=True)
    o = _conv3x3_s1(o, w2, b2, relu=False)
    idn = _down1x1_s2_val(x, wd, bd)
    return jnp.maximum(o + idn, 0.0)


def _block_s1(x, w1, b1, w2, b2, mask=None):
    o = _conv3x3_s1(x, w1, b1, relu=True, mask=mask)
    o = _conv3x3_s1(o, w2, b2, relu=False, mask=mask)
    return jnp.maximum(o + x, 0.0)


def _block_s2(x, sref, w1, b1, w2, b2, wd, bd):
    _store_padded(sref, x)
    o = _conv3x3_s2(sref, w1, b1, relu=True)
    o = _conv3x3_s1(o, w2, b2, relu=False)
    idn = _down1x1_s2(sref, wd, bd)
    return jnp.maximum(o + idn, 0.0)


def _maxpool3x3_s2(xp_ref):
    bm, Hp, Wp, C = xp_ref.shape
    Ho, Wo = (Hp - 2) // 2, (Wp - 2) // 2
    out = None
    for i in range(3):
        for j in range(3):
            t = xp_ref[:, pl.ds(i, Ho, 2), pl.ds(j, Wo, 2), :]
            out = t if out is None else jnp.maximum(out, t)
    return out.astype(jnp.bfloat16)


def _front_kernel(img_ref, c1w, c1b, c1m,
                  a0c1w, a0c1b, a0c2w, a0c2b, a1c1w, a1c1b, a1c2w, a1c2b,
                  b0c1w, b0c1b, b0c2w, b0c2b, b0dw, b0db,
                  b1c1w, b1c1b, b1c2w, b1c2b, o_ref, sp_ref, s1_ref, *, bm):
    x = img_ref[...]
    pieces = [x[:, dY:dY + 32, dG:dG + 8, :] for dY in range(4) for dG in range(2)]
    xi = jnp.concatenate(pieces, axis=-1).reshape(bm * 256, 384)
    acc = jnp.dot(xi, c1w[...], preferred_element_type=jnp.float32)
    acc = jnp.maximum(acc + c1b[...], 0.0) * c1m[...]
    x = acc.astype(jnp.bfloat16).reshape(bm, 32, 32, 128)
    _store_padded(sp_ref, x)
    x = _maxpool3x3_s2(sp_ref)
    h = bm // 2
    x5 = x.reshape(h, 2, 16, 16, 128)
    x = jnp.concatenate([x5[:, 0, :, :, 0:64], x5[:, 1, :, :, 0:64]], axis=-1)
    x = _block_s1(x, a0c1w, a0c1b, a0c2w, a0c2b)
    x = _block_s1(x, a1c1w, a1c1b, a1c2w, a1c2b)
    xs = jnp.stack([x[:, :, :, 0:64], x[:, :, :, 64:128]], axis=1)
    x = xs.reshape(bm, 16, 16, 64)
    x = jnp.concatenate([x, jnp.zeros((bm, 16, 16, 64), jnp.bfloat16)], axis=-1)
    x = _block_s2(x, s1_ref, b0c1w, b0c1b, b0c2w, b0c2b, b0dw, b0db)
    x = _block_s1(x, b1c1w, b1c1b, b1c2w, b1c2b)
    o_ref[...] = x


def _back_kernel(x_ref, auto_ref, pos_ref,
                 c0c1w, c0c1b, c0c2w, c0c2b, c0dw, c0db,
                 c1c1w, c1c1b, c1c2w, c1c2b,
                 d0c1w, d0c1b, d0c2w, d0c2b, d0dw, d0db,
                 d1c1w, d1c1b, d1c2w, d1c2b,
                 fc0w, fc0b, fc1w, fc1b,
                 p0w, p0b, p1w, p1b, p2w, p2b, p3w, p3b, p4w, p4b,
                 aw, ab, dw, db, o_ref, s2_ref, *, bm):
    x = x_ref[...]
    x = _block_s2(x, s2_ref, c0c1w, c0c1b, c0c2w, c0c2b, c0dw, c0db)
    x = _block_s1(x, c1c1w, c1c1b, c1c2w, c1c2b)
    x = _block_s2_val(x, d0c1w, d0c1b, d0c2w, d0c2b, d0dw, d0db)
    x = _block_s1(x, d1c1w, d1c1b, d1c2w, d1c2b)
    g = jnp.mean(x.astype(jnp.float32), axis=(1, 2))
    h = g.astype(jnp.bfloat16)
    a = jnp.maximum(jnp.dot(h, fc0w[...], preferred_element_type=jnp.float32)
                    + fc0b[...], 0.0)
    img = jnp.maximum(jnp.dot(a.astype(jnp.bfloat16), fc1w[...],
                              preferred_element_type=jnp.float32) + fc1b[...], 0.0)
    h = pos_ref[...].astype(jnp.bfloat16)
    for w_r, b_r in ((p0w, p0b), (p1w, p1b), (p2w, p2b), (p3w, p3b), (p4w, p4b)):
        acc = jnp.maximum(jnp.dot(h, w_r[...], preferred_element_type=jnp.float32)
                          + b_r[...], 0.0)
        h = acc.astype(jnp.bfloat16)
    pos = acc
    a = jnp.maximum(jnp.dot(auto_ref[...].astype(jnp.bfloat16), aw[...],
                            preferred_element_type=jnp.float32) + ab[...], 0.0)
    enc = a + pos + img
    d = jnp.dot(enc.astype(jnp.bfloat16), dw[...], preferred_element_type=jnp.float32)
    o_ref[...] = jnp.maximum(d + db[...], 0.0)


def kernel(gene_info, spot_position_info, spot_image, noise_key, auto_w, auto_b,
           dec_w, dec_b, pos0_w, pos0_b, pos1_w, pos1_b, pos2_w, pos2_b,
           pos3_w, pos3_b, pos4_w, pos4_b, fc0_w, fc0_b, fc1_w, fc1_b,
           conv1_w, conv1_b,
           L0b0_c1_w, L0b0_c1_b, L0b0_c2_w, L0b0_c2_b,
           L0b1_c1_w, L0b1_c1_b, L0b1_c2_w, L0b1_c2_b,
           L1b0_c1_w, L1b0_c1_b, L1b0_c2_w, L1b0_c2_b, L1b0_d_w, L1b0_d_b,
           L1b1_c1_w, L1b1_c1_b, L1b1_c2_w, L1b1_c2_b,
           L2b0_c1_w, L2b0_c1_b, L2b0_c2_w, L2b0_c2_b, L2b0_d_w, L2b0_d_b,
           L2b1_c1_w, L2b1_c1_b, L2b1_c2_w, L2b1_c2_b,
           L3b0_c1_w, L3b0_c1_b, L3b0_c2_w, L3b0_c2_b, L3b0_d_w, L3b0_d_b,
           L3b1_c1_w, L3b1_c1_b, L3b1_c2_w, L3b1_c2_b):
    B = gene_info.shape[0]

    gmean = jnp.mean(gene_info)
    gstd = jnp.std(gene_info, ddof=1)
    noise = jax.random.normal(noise_key, gene_info.shape, jnp.float32) * gstd + gmean
    auto_in = gene_info + jnp.maximum(noise * 0.1, 0.0)

    xp = jnp.pad(spot_image.astype(jnp.bfloat16),
                 ((0, 0), (0, 0), (3, 3), (3, 5)))
    xr = xp.reshape(B, 3, 35, 2, 9, 8).transpose(0, 2, 4, 3, 5, 1)
    xr = xr.reshape(B, 35, 9, 48)

    w7 = conv1_w.reshape(7, 7, 3, 128)
    wjp = jnp.pad(w7, ((0, 1), (6, 9), (0, 0), (0, 0)))
    dYv = jnp.arange(4)[:, None, None, None]
    dGv = jnp.arange(2)[None, :, None, None]
    phrv = jnp.arange(2)[None, None, :, None]
    phcv = jnp.arange(8)[None, None, None, :]
    i_idx = jnp.broadcast_to(2 * dYv + phrv, (4, 2, 2, 8))
    w_u = [wjp[i_idx, jnp.broadcast_to(8 * dGv + phcv - 2 * u + 6, (4, 2, 2, 8))]
           .reshape(384, 128) for u in range(4)]
    c1w384 = jnp.stack(w_u, axis=1).reshape(384, 512)
    c1b512 = jnp.tile(conv1_b.reshape(1, 128).astype(jnp.float32), (1, 4))
    c1m512 = ((jnp.arange(512) % 128) < 64).astype(jnp.float32).reshape(1, 512)

    def cinpad(w):
        return jnp.pad(w.reshape(3, 3, 64, -1),
                       ((0, 0), (0, 0), (0, 64), (0, 0))).reshape(1152, -1)

    def rb(b):
        return b.reshape(1, -1).astype(jnp.float32)

    def pack_w(w):
        wsm = w.reshape(3, 3, 64, 128)[:, :, :, 0:64]
        z = jnp.zeros_like(wsm)
        top = jnp.concatenate([wsm, z], axis=-1)
        bot = jnp.concatenate([z, wsm], axis=-1)
        return jnp.concatenate([top, bot], axis=2).reshape(1152, 128)

    def pack_b(b):
        b64 = b.reshape(1, -1)[:, 0:64].astype(jnp.float32)
        return jnp.concatenate([b64, b64], axis=-1)

    bm1 = 16 if B % 16 == 0 else B
    f_in = [xr, c1w384, c1b512, c1m512,
            pack_w(L0b0_c1_w), pack_b(L0b0_c1_b),
            pack_w(L0b0_c2_w), pack_b(L0b0_c2_b),
            pack_w(L0b1_c1_w), pack_b(L0b1_c1_b),
            pack_w(L0b1_c2_w), pack_b(L0b1_c2_b),
            cinpad(L1b0_c1_w), rb(L1b0_c1_b), L1b0_c2_w, rb(L1b0_c2_b),
            jnp.pad(L1b0_d_w, ((0, 64), (0, 0))), rb(L1b0_d_b),
            L1b1_c1_w, rb(L1b1_c1_b), L1b1_c2_w, rb(L1b1_c2_b)]
    front = pl.pallas_call(
        functools.partial(_front_kernel, bm=bm1),
        out_shape=jax.ShapeDtypeStruct((B, 8, 8, 128), jnp.bfloat16),
        grid=(B // bm1,),
        in_specs=[pl.BlockSpec((bm1, 35, 9, 48), lambda i: (i, 0, 0, 0))]
                 + [_wspec(a.shape) for a in f_in[1:]],
        out_specs=pl.BlockSpec((bm1, 8, 8, 128), lambda i: (i, 0, 0, 0)),
        scratch_shapes=[pltpu.VMEM((bm1, 34, 34, 128), jnp.float32),
                        pltpu.VMEM((bm1, 18, 18, 128), jnp.float32)],
        compiler_params=pltpu.CompilerParams(dimension_semantics=("parallel",)),
    )(*f_in)

    bm2 = 128 if B % 128 == 0 else B
    b_in = [front, auto_in, spot_position_info,
            L2b0_c1_w, rb(L2b0_c1_b), L2b0_c2_w, rb(L2b0_c2_b),
            L2b0_d_w, rb(L2b0_d_b),
            L2b1_c1_w, rb(L2b1_c1_b), L2b1_c2_w, rb(L2b1_c2_b),
            L3b0_c1_w, rb(L3b0_c1_b), L3b0_c2_w, rb(L3b0_c2_b),
            L3b0_d_w, rb(L3b0_d_b),
            L3b1_c1_w, rb(L3b1_c1_b), L3b1_c2_w, rb(L3b1_c2_b),
            fc0_w, rb(fc0_b), fc1_w, rb(fc1_b),
            pos0_w, rb(pos0_b), pos1_w, rb(pos1_b), pos2_w, rb(pos2_b),
            pos3_w, rb(pos3_b), pos4_w, rb(pos4_b),
            auto_w, rb(auto_b), dec_w, rb(dec_b)]
    out = pl.pallas_call(
        functools.partial(_back_kernel, bm=bm2),
        out_shape=jax.ShapeDtypeStruct((B, 256), jnp.float32),
        grid=(B // bm2,),
        in_specs=[pl.BlockSpec((bm2, 8, 8, 128), lambda i: (i, 0, 0, 0)),
                  pl.BlockSpec((bm2, 256), lambda i: (i, 0)),
                  pl.BlockSpec((bm2, 8), lambda i: (i, 0))]
                 + [_wspec(a.shape) for a in b_in[3:]],
        out_specs=pl.BlockSpec((bm2, 256), lambda i: (i, 0)),
        scratch_shapes=[pltpu.VMEM((bm2, 10, 10, 128), jnp.float32)],
        compiler_params=pltpu.CompilerParams(dimension_semantics=("parallel",)),
    )(*b_in)
    return out
